# baseline (speedup 1.0000x reference)
import functools

import jax
import jax.numpy as jnp
from jax import lax
from jax.experimental import pallas as pl
from jax.experimental.pallas import tpu as pltpu
from jax.experimental.pallas import tpu_sc as plsc

_BATCH = 4096
_SEQ = 200
_D = 32
_N = _BATCH * _SEQ
_NW = 32
_BLK = _BATCH // _NW
_PER_W = _SEQ * _BLK

_mesh = plsc.VectorSubcoreMesh(core_axis_name="c", subcore_axis_name="s")


@functools.partial(
    pl.kernel,
    mesh=_mesh,
    out_type=jax.ShapeDtypeStruct((_SEQ, _D, _BATCH), jnp.float32),
    scratch_types=[
        pltpu.VMEM((_PER_W,), jnp.int32),
        pltpu.VMEM((_PER_W,), jnp.int32),
        pltpu.VMEM((_BLK, _D), jnp.float32),
        pltpu.VMEM((_BLK, _D), jnp.float32),
        pltpu.VMEM((_D, _BLK), jnp.float32),
        pltpu.VMEM((_D, _BLK), jnp.float32),
        pltpu.SemaphoreType.DMA,
        pltpu.SemaphoreType.DMA,
        pltpu.SemaphoreType.DMA,
        pltpu.SemaphoreType.DMA,
        pltpu.SemaphoreType.DMA,
    ],
    compiler_params=pltpu.CompilerParams(use_tc_tiling_on_sc=False,
                                         needs_layout_passes=False,
                                         disable_bounds_checks=True),
)
def _double_gather(ids_hbm, remap_hbm, emb_hbm, out_hbm,
                   ids_v, gids_v, r0, r1, t0, t1,
                   sem_a, sg0, sg1, ss0, ss1):
    wid = lax.axis_index("s") * 2 + lax.axis_index("c")
    b0 = wid * _BLK
    lanes = lax.iota(jnp.int32, 16)

    def ids_row(s, carry):
        pltpu.async_copy(ids_hbm.at[s, pl.ds(b0, _BLK)],
                         ids_v.at[pl.ds(s * _BLK, _BLK)], sem_a)
        return carry

    lax.fori_loop(0, _SEQ, ids_row, 0)

    def ids_row_wait(s, carry):
        pltpu.make_async_copy(ids_hbm.at[0, pl.ds(b0, _BLK)],
                              ids_v.at[pl.ds(0, _BLK)], sem_a).wait()
        return carry

    lax.fori_loop(0, _SEQ, ids_row_wait, 0)
    pltpu.async_copy(remap_hbm.at[ids_v], gids_v, sem_a).wait()

    def g2_start(s, buf, sem):
        pltpu.async_copy(emb_hbm.at[gids_v.at[pl.ds(s * _BLK, _BLK)]],
                         buf, sem)

    def g2_wait(buf, sem):
        pltpu.make_async_copy(emb_hbm.at[gids_v.at[pl.ds(0, _BLK)]], buf,
                              sem).wait()

    def st_start(s, buf, sem):
        pltpu.async_copy(buf, out_hbm.at[s, :, pl.ds(b0, _BLK)], sem)

    def st_wait(buf, sem):
        pltpu.make_async_copy(buf, out_hbm.at[0, :, pl.ds(b0, _BLK)],
                              sem).wait()

    bvecs = [b16 * 16 + lanes for b16 in range(_BLK // 16)]

    def transpose(rbuf, tbuf):
        pend = None
        for b16 in range(_BLK // 16):
            bvec = bvecs[b16]
            for k8 in range(_D // 16):
                ms = [(bvec + (k8 * 16 + i)) & (_D - 1) for i in range(16)]
                vals = [plsc.load_gather(rbuf, [bvec, m]) for m in ms]
                if pend is not None:
                    for pm, pb, pv in pend:
                        plsc.store_scatter(tbuf, [pm, pb], pv)
                pend = [(m, bvec, v) for m, v in zip(ms, vals)]
        for pm, pb, pv in pend:
            plsc.store_scatter(tbuf, [pm, pb], pv)

    g2_start(0, r0, sg0)

    def pair(j, carry):
        s0 = 2 * j
        g2_start(s0 + 1, r1, sg1)
        g2_wait(r0, sg0)

        @pl.when(j > 0)
        def _():
            st_wait(t0, ss0)
        transpose(r0, t0)
        st_start(s0, t0, ss0)

        @pl.when(j < _SEQ // 2 - 1)
        def _():
            g2_start(s0 + 2, r0, sg0)
        g2_wait(r1, sg1)

        @pl.when(j > 0)
        def _():
            st_wait(t1, ss1)
        transpose(r1, t1)
        st_start(s0 + 1, t1, ss1)
        return carry

    lax.fori_loop(0, _SEQ // 2, pair, 0)
    st_wait(t0, ss0)
    st_wait(t1, ss1)


def kernel(client_ids, item_ids, item_id2graph_id, item_embeddings):
    del client_ids
    out = _double_gather(item_ids.T.astype(jnp.int32),
                         item_id2graph_id.astype(jnp.int32),
                         item_embeddings)
    return out.transpose(2, 0, 1)


# speedup vs baseline: 1.1380x; 1.1380x over previous
import functools

import jax
import jax.numpy as jnp
from jax import lax
from jax.experimental import pallas as pl
from jax.experimental.pallas import tpu as pltpu
from jax.experimental.pallas import tpu_sc as plsc

_BATCH = 4096
_SEQ = 200
_D = 32
_N = _BATCH * _SEQ
_NW = 32
_BLK = _BATCH // _NW
_PER_W = _SEQ * _BLK

_mesh = plsc.VectorSubcoreMesh(core_axis_name="c", subcore_axis_name="s")


@functools.partial(
    pl.kernel,
    mesh=_mesh,
    out_type=jax.ShapeDtypeStruct((_SEQ, _D, _BATCH), jnp.float32),
    scratch_types=[
        pltpu.VMEM((_PER_W,), jnp.int32),
        pltpu.VMEM((_PER_W,), jnp.int32),
        pltpu.VMEM((_BLK, _D), jnp.float32),
        pltpu.VMEM((_BLK, _D), jnp.float32),
        pltpu.VMEM((_D, _BLK), jnp.float32),
        pltpu.VMEM((_D, _BLK), jnp.float32),
        pltpu.SemaphoreType.DMA,
        pltpu.SemaphoreType.DMA,
        pltpu.SemaphoreType.DMA,
        pltpu.SemaphoreType.DMA,
        pltpu.SemaphoreType.DMA,
    ],
    compiler_params=pltpu.CompilerParams(use_tc_tiling_on_sc=False,
                                         needs_layout_passes=False,
                                         disable_bounds_checks=True),
)
def _double_gather(ids_hbm, remap_hbm, emb_hbm, out_hbm,
                   ids_v, gids_v, r0, r1, t0, t1,
                   sem_a, sg0, sg1, ss0, ss1):
    wid = lax.axis_index("s") * 2 + lax.axis_index("c")
    b0 = wid * _BLK
    lanes = lax.iota(jnp.int32, 16)

    def ids_row(s, carry):
        pltpu.async_copy(ids_hbm.at[s, pl.ds(b0, _BLK)],
                         ids_v.at[pl.ds(s * _BLK, _BLK)], sem_a)
        return carry

    lax.fori_loop(0, _SEQ, ids_row, 0)

    def ids_row_wait(s, carry):
        pltpu.make_async_copy(ids_hbm.at[0, pl.ds(b0, _BLK)],
                              ids_v.at[pl.ds(0, _BLK)], sem_a).wait()
        return carry

    lax.fori_loop(0, _SEQ, ids_row_wait, 0)
    pltpu.async_copy(remap_hbm.at[ids_v], gids_v, sem_a).wait()

    def g2_start(s, buf, sem):
        pltpu.async_copy(emb_hbm.at[gids_v.at[pl.ds(s * _BLK, _BLK)]],
                         buf, sem)

    def g2_wait(buf, sem):
        pltpu.make_async_copy(emb_hbm.at[gids_v.at[pl.ds(0, _BLK)]], buf,
                              sem).wait()

    def st_start(s, buf, sem):
        pltpu.async_copy(buf, out_hbm.at[s, :, pl.ds(b0, _BLK)], sem)

    def st_wait(buf, sem):
        pltpu.make_async_copy(buf, out_hbm.at[0, :, pl.ds(b0, _BLK)],
                              sem).wait()

    bvecs = [b16 * 16 + lanes for b16 in range(_BLK // 16)]

    def transpose(rbuf, tbuf):
        for b16 in range(_BLK // 16):
            bvec = bvecs[b16]
            for k8 in range(_D // 16):
                ms = [(bvec + (k8 * 16 + i)) & (_D - 1) for i in range(16)]
                vals = [plsc.load_gather(rbuf, [bvec, m]) for m in ms]
                for m, v in zip(ms, vals):
                    plsc.store_scatter(tbuf, [m, bvec], v)

    g2_start(0, r0, sg0)

    def pair(j, carry):
        s0 = 2 * j
        g2_start(s0 + 1, r1, sg1)
        g2_wait(r0, sg0)

        @pl.when(j > 0)
        def _():
            st_wait(t0, ss0)
        transpose(r0, t0)
        st_start(s0, t0, ss0)

        @pl.when(j < _SEQ // 2 - 1)
        def _():
            g2_start(s0 + 2, r0, sg0)
        g2_wait(r1, sg1)

        @pl.when(j > 0)
        def _():
            st_wait(t1, ss1)
        transpose(r1, t1)
        st_start(s0 + 1, t1, ss1)
        return carry

    lax.fori_loop(0, _SEQ // 2, pair, 0)
    st_wait(t0, ss0)
    st_wait(t1, ss1)


def kernel(client_ids, item_ids, item_id2graph_id, item_embeddings):
    del client_ids
    out = _double_gather(item_ids.T.astype(jnp.int32),
                         item_id2graph_id.astype(jnp.int32),
                         item_embeddings)
    return out.transpose(2, 0, 1)
